# baseline (speedup 1.0000x reference)
import jax
import jax.numpy as jnp
from jax import lax
from jax.experimental import pallas as pl
from jax.experimental.pallas import tpu as pltpu
from jax.experimental.pallas import tpu_sc as plsc

VOCAB = 100000
HIDDEN = 4096
TOKENS = 8192

NC = 2
NS = 16
NW = NC * NS
TOK_PER_W = TOKENS // NW
C = 8
NCHUNK = TOK_PER_W // C

_mesh = plsc.VectorSubcoreMesh(
    core_axis_name="c", subcore_axis_name="s", num_cores=NC, num_subcores=NS
)


def _embed(idx, weight):
    def body(idx_hbm, table_hbm, out_hbm, idx_v, buf0, buf1,
             gsem0, gsem1, ssem0, ssem1):
        wid = lax.axis_index("s") * NC + lax.axis_index("c")
        base = wid * TOK_PER_W
        pltpu.sync_copy(idx_hbm.at[pl.ds(base, TOK_PER_W)], idx_v)

        bufs = (buf0, buf1)
        gsems = (gsem0, gsem1)
        ssems = (ssem0, ssem1)

        def gather_desc(j, b):
            return pltpu.make_async_copy(
                table_hbm.at[idx_v.at[pl.ds(j * C, C)]], bufs[b], gsems[b])

        def scatter_desc(j, b):
            return pltpu.make_async_copy(
                bufs[b], out_hbm.at[pl.ds(base + j * C, C)], ssems[b])

        gather_desc(0, 0).start()

        @pl.loop(0, NCHUNK // 2)
        def _(g):
            j0 = 2 * g
            @pl.when(g > 0)
            def _():
                scatter_desc(j0 - 1, 1).wait()
            gather_desc(j0 + 1, 1).start()
            gather_desc(j0, 0).wait()
            scatter_desc(j0, 0).start()
            scatter_desc(j0, 0).wait()

            @pl.when(g < NCHUNK // 2 - 1)
            def _():
                gather_desc(j0 + 2, 0).start()
            gather_desc(j0 + 1, 1).wait()
            scatter_desc(j0 + 1, 1).start()

        scatter_desc(NCHUNK - 1, 1).wait()

    f = pl.kernel(
        body,
        out_type=jax.ShapeDtypeStruct((TOKENS, HIDDEN), jnp.float32),
        mesh=_mesh,
        scratch_types=[
            pltpu.VMEM((TOK_PER_W,), jnp.int32),
            pltpu.VMEM((C, HIDDEN), jnp.float32),
            pltpu.VMEM((C, HIDDEN), jnp.float32),
            pltpu.SemaphoreType.DMA,
            pltpu.SemaphoreType.DMA,
            pltpu.SemaphoreType.DMA,
            pltpu.SemaphoreType.DMA,
        ],
    )
    return f(idx, weight)


def kernel(input, weight):
    return _embed(input, weight)
